# baseline (speedup 1.0000x reference)
import functools

import jax
import jax.numpy as jnp
from jax import lax
from jax.experimental import pallas as pl
from jax.experimental.pallas import tpu as pltpu
from jax.experimental.pallas import tpu_sc as plsc

ROWS = 4096
COLS = 8192
GROUPS = 64
PAIR_W = 256
LANES = 16

NUM_CORES = 2
NUM_SUBCORES = 16
NUM_WORKERS = NUM_CORES * NUM_SUBCORES
ROWS_PER_W = ROWS // NUM_WORKERS

ROW_TILE = LANES
ROW_TILES = ROWS_PER_W // ROW_TILE
CHUNK_COLS = 2048
COL_CHUNKS = COLS // CHUNK_COLS
NCHUNKS = ROW_TILES * COL_CHUNKS
PAIRS_PER_CHUNK = CHUNK_COLS // PAIR_W
GROUPS_PER_CHUNK = 2 * PAIRS_PER_CHUNK

PAD_COLS = CHUNK_COLS + 8
BUF_WORDS = ROW_TILE * PAD_COLS
GATHER_SPAN = 15 * PAD_COLS + 8
NACC = 2


def _tree_sum(vecs):
    while len(vecs) > 1:
        nxt = [vecs[i] + vecs[i + 1] for i in range(0, len(vecs) - 1, 2)]
        if len(vecs) % 2:
            nxt.append(vecs[-1])
        vecs = nxt
    return vecs[0]


@functools.partial(
    pl.kernel,
    out_type=jax.ShapeDtypeStruct((ROWS * GROUPS,), jnp.float32),
    mesh=plsc.VectorSubcoreMesh(core_axis_name="c", subcore_axis_name="s"),
    scratch_types=[
        pltpu.VMEM((BUF_WORDS,), jnp.float32),
        pltpu.VMEM((BUF_WORDS,), jnp.float32),
        pltpu.VMEM((ROWS_PER_W * GROUPS,), jnp.float32),
        pltpu.SemaphoreType.DMA,
        pltpu.SemaphoreType.DMA,
    ],
    compiler_params=pltpu.CompilerParams(needs_layout_passes=False),
)
def _sc_group_sum(x_hbm, out_hbm, buf0, buf1, out_v, sem0, sem1):
    wid = lax.axis_index("s") * NUM_CORES + lax.axis_index("c")
    base_row = wid * ROWS_PER_W

    def start_chunk(c, buf, sem):
        rt = c // COL_CHUNKS
        cc = c % COL_CHUNKS
        for l in range(ROW_TILE):
            src = (base_row + rt * ROW_TILE + l) * COLS + cc * CHUNK_COLS
            pltpu.async_copy(
                x_hbm.at[pl.ds(src, CHUNK_COLS)],
                buf.at[pl.ds(l * PAD_COLS, CHUNK_COLS)],
                sem)

    def wait_chunk(buf, sem):
        for l in range(ROW_TILE):
            pltpu.make_async_copy(
                x_hbm.at[pl.ds(0, CHUNK_COLS)],
                buf.at[pl.ds(l * PAD_COLS, CHUNK_COLS)],
                sem).wait()

    start_chunk(0, buf0, sem0)
    start_chunk(1, buf1, sem1)

    lane_iota = lax.iota(jnp.int32, LANES)
    row_off = lane_iota * PAD_COLS
    out_iota = lane_iota * GROUPS
    idx_phase = [row_off + m for m in range(8)]

    def group_sum(buf, b, width):
        accs = [None] * NACC
        n = 0
        for j in range(width // 8):
            sl = buf.at[pl.ds(b + 8 * j, GATHER_SPAN)]
            for m in range(8):
                v = plsc.load_gather(sl, [idx_phase[m]])
                k = n % NACC
                accs[k] = v if accs[k] is None else accs[k] + v
                n += 1
        return _tree_sum(accs)

    def make_pair_body(buf, r0, g0):
        def pair_body(p, carry):
            b = p * PAIR_W
            for parity, w in ((0, 64), (1, 192)):
                s = group_sum(buf, b + parity * 64, w)
                dst = out_iota + lax.broadcast_in_dim(
                    r0 * GROUPS + g0 + 2 * p + parity, (LANES,), ())
                plsc.store_scatter(out_v, [dst], s)
            return carry
        return pair_body

    def chunk_pair_body(i, carry):
        for off, buf, sem in ((0, buf0, sem0), (1, buf1, sem1)):
            c = 2 * i + off
            wait_chunk(buf, sem)
            r0 = (c // COL_CHUNKS) * ROW_TILE
            g0 = (c % COL_CHUNKS) * GROUPS_PER_CHUNK
            lax.fori_loop(0, PAIRS_PER_CHUNK, make_pair_body(buf, r0, g0), 0)
            @pl.when(c + 2 < NCHUNKS)
            def _():
                start_chunk(c + 2, buf, sem)
        return carry

    lax.fori_loop(0, NCHUNKS // 2, chunk_pair_body, 0)

    pltpu.sync_copy(
        out_v,
        out_hbm.at[pl.ds(base_row * GROUPS, ROWS_PER_W * GROUPS)])


def kernel(x):
    y = _sc_group_sum(x.reshape(-1))
    return y.reshape(ROWS, GROUPS)


# speedup vs baseline: 1.8579x; 1.8579x over previous
import functools

import jax
import jax.numpy as jnp
from jax import lax
from jax.experimental import pallas as pl
from jax.experimental.pallas import tpu as pltpu
from jax.experimental.pallas import tpu_sc as plsc

ROWS = 4096
COLS = 8192
GROUPS = 64
PAIR_W = 256
PAIRS = 32
LANES = 16

NUM_CORES = 2
NUM_SUBCORES = 16
NUM_WORKERS = NUM_CORES * NUM_SUBCORES

SC_ROWS = 1536
TC_ROWS = ROWS - SC_ROWS
ROWS_PER_W = SC_ROWS // NUM_WORKERS

CHUNK_ROWS = 4
NCHUNKS = ROWS_PER_W // CHUNK_ROWS

TBR = 256


def _tree_sum(vecs):
    while len(vecs) > 1:
        nxt = [vecs[i] + vecs[i + 1] for i in range(0, len(vecs) - 1, 2)]
        if len(vecs) % 2:
            nxt.append(vecs[-1])
        vecs = nxt
    return vecs[0]


@functools.partial(
    pl.kernel,
    out_type=jax.ShapeDtypeStruct((SC_ROWS, GROUPS), jnp.float32),
    mesh=plsc.VectorSubcoreMesh(core_axis_name="c", subcore_axis_name="s"),
    scratch_types=[
        pltpu.VMEM((CHUNK_ROWS, COLS), jnp.float32),
        pltpu.VMEM((CHUNK_ROWS, COLS), jnp.float32),
        pltpu.VMEM((ROWS_PER_W, GROUPS), jnp.float32),
        pltpu.SemaphoreType.DMA,
        pltpu.SemaphoreType.DMA,
    ],
    compiler_params=pltpu.CompilerParams(needs_layout_passes=False),
)
def _sc_group_sum(x_hbm, out_hbm, buf0, buf1, out_v, sem0, sem1):
    wid = lax.axis_index("s") * NUM_CORES + lax.axis_index("c")
    base_row = wid * ROWS_PER_W

    pltpu.async_copy(x_hbm.at[pl.ds(base_row, CHUNK_ROWS)], buf0, sem0)
    pltpu.async_copy(
        x_hbm.at[pl.ds(base_row + CHUNK_ROWS, CHUNK_ROWS)], buf1, sem1)

    lane_iota = lax.iota(jnp.int32, LANES)
    lane_eq = [
        lane_iota == lax.broadcast_in_dim(jnp.int32(j), (LANES,), ())
        for j in range(LANES)
    ]

    def make_row_body(buf, chunk_idx):
        def row_body(r, carry):
            out_r = chunk_idx * CHUNK_ROWS + r
            for q in range(GROUPS // LANES):
                acc = None
                for j in range(LANES):
                    g = q * LANES + j
                    p, odd = divmod(g, 2)
                    b = p * PAIR_W + odd * 64
                    nvec = 12 if odd else 4
                    v = None
                    for c0 in range(0, nvec, 4):
                        t = _tree_sum(
                            [buf[r, pl.ds(b + (c0 + k) * LANES, LANES)]
                             for k in range(4)])
                        v = t if v is None else v + t
                    s = lax.broadcast_in_dim(jnp.sum(v), (LANES,), ())
                    acc = s if acc is None else jnp.where(lane_eq[j], s, acc)
                out_v[out_r, pl.ds(q * LANES, LANES)] = acc
            return carry
        return row_body

    def pair_body(i, carry):
        for off, buf, sem in ((0, buf0, sem0), (1, buf1, sem1)):
            c = 2 * i + off
            pltpu.make_async_copy(
                x_hbm.at[pl.ds(0, CHUNK_ROWS)], buf, sem).wait()
            lax.fori_loop(0, CHUNK_ROWS, make_row_body(buf, c), 0)
            @pl.when(c + 2 < NCHUNKS)
            def _():
                pltpu.async_copy(
                    x_hbm.at[pl.ds(base_row + (c + 2) * CHUNK_ROWS,
                                   CHUNK_ROWS)],
                    buf, sem)
        return carry

    lax.fori_loop(0, NCHUNKS // 2, pair_body, 0)

    pltpu.sync_copy(out_v, out_hbm.at[pl.ds(base_row, ROWS_PER_W)])


def _tc_body(x_ref, m_ref, o_ref):
    o_ref[...] = jnp.dot(
        x_ref[...], m_ref[...], preferred_element_type=jnp.float32,
        precision=lax.Precision.HIGHEST)


def _tc_group_sum(x, m):
    return pl.pallas_call(
        _tc_body,
        grid=((ROWS - SC_ROWS) // TBR,),
        in_specs=[
            pl.BlockSpec((TBR, COLS), lambda i: (i + SC_ROWS // TBR, 0)),
            pl.BlockSpec((COLS, GROUPS), lambda i: (0, 0)),
        ],
        out_specs=pl.BlockSpec((TBR, GROUPS), lambda i: (i, 0)),
        out_shape=jax.ShapeDtypeStruct((ROWS - SC_ROWS, GROUPS), jnp.float32),
    )(x, m)


def kernel(x):
    col = jnp.arange(COLS)
    gid = 2 * (col // PAIR_W) + (col % PAIR_W >= 64).astype(jnp.int32)
    m = (gid[:, None] == jnp.arange(GROUPS)[None, :]).astype(jnp.float32)
    y_sc = _sc_group_sum(x)
    y_tc = _tc_group_sum(x, m)
    return jnp.concatenate([y_sc, y_tc], axis=0)


# speedup vs baseline: 2.7042x; 1.4555x over previous
import functools

import jax
import jax.numpy as jnp
from jax import lax
from jax.experimental import pallas as pl
from jax.experimental.pallas import tpu as pltpu
from jax.experimental.pallas import tpu_sc as plsc

ROWS = 4096
COLS = 8192
GROUPS = 64
PAIR_W = 256
PAIRS = 32
LANES = 16

NUM_CORES = 2
NUM_SUBCORES = 16
NUM_WORKERS = NUM_CORES * NUM_SUBCORES

SC_ROWS = 2048
TC_ROWS = ROWS - SC_ROWS
ROWS_PER_W = SC_ROWS // NUM_WORKERS

CHUNK_ROWS = 4
NCHUNKS = ROWS_PER_W // CHUNK_ROWS

TBR = 256


def _tree_sum(vecs):
    while len(vecs) > 1:
        nxt = [vecs[i] + vecs[i + 1] for i in range(0, len(vecs) - 1, 2)]
        if len(vecs) % 2:
            nxt.append(vecs[-1])
        vecs = nxt
    return vecs[0]


@functools.partial(
    pl.kernel,
    out_type=jax.ShapeDtypeStruct((SC_ROWS, GROUPS), jnp.float32),
    mesh=plsc.VectorSubcoreMesh(core_axis_name="c", subcore_axis_name="s"),
    scratch_types=[
        pltpu.VMEM((CHUNK_ROWS, COLS), jnp.float32),
        pltpu.VMEM((CHUNK_ROWS, COLS), jnp.float32),
        pltpu.VMEM((ROWS_PER_W, GROUPS), jnp.float32),
        pltpu.SemaphoreType.DMA,
        pltpu.SemaphoreType.DMA,
    ],
    compiler_params=pltpu.CompilerParams(needs_layout_passes=False),
)
def _sc_group_sum(x_hbm, out_hbm, buf0, buf1, out_v, sem0, sem1):
    wid = lax.axis_index("s") * NUM_CORES + lax.axis_index("c")
    base_row = wid * ROWS_PER_W

    pltpu.async_copy(x_hbm.at[pl.ds(base_row, CHUNK_ROWS)], buf0, sem0)
    pltpu.async_copy(
        x_hbm.at[pl.ds(base_row + CHUNK_ROWS, CHUNK_ROWS)], buf1, sem1)

    lane_iota = lax.iota(jnp.int32, LANES)
    lane_eq = [
        lane_iota == lax.broadcast_in_dim(jnp.int32(j), (LANES,), ())
        for j in range(LANES)
    ]

    def make_row_body(buf, chunk_idx):
        def row_body(r, carry):
            out_r = chunk_idx * CHUNK_ROWS + r
            for q in range(GROUPS // LANES):
                acc = None
                for j in range(LANES):
                    g = q * LANES + j
                    p, odd = divmod(g, 2)
                    b = p * PAIR_W + odd * 64
                    nvec = 12 if odd else 4
                    v = None
                    for c0 in range(0, nvec, 4):
                        t = _tree_sum(
                            [buf[r, pl.ds(b + (c0 + k) * LANES, LANES)]
                             for k in range(4)])
                        v = t if v is None else v + t
                    s = lax.broadcast_in_dim(jnp.sum(v), (LANES,), ())
                    acc = s if acc is None else jnp.where(lane_eq[j], s, acc)
                out_v[out_r, pl.ds(q * LANES, LANES)] = acc
            return carry
        return row_body

    def pair_body(i, carry):
        for off, buf, sem in ((0, buf0, sem0), (1, buf1, sem1)):
            c = 2 * i + off
            pltpu.make_async_copy(
                x_hbm.at[pl.ds(0, CHUNK_ROWS)], buf, sem).wait()
            lax.fori_loop(0, CHUNK_ROWS, make_row_body(buf, c), 0)
            @pl.when(c + 2 < NCHUNKS)
            def _():
                pltpu.async_copy(
                    x_hbm.at[pl.ds(base_row + (c + 2) * CHUNK_ROWS,
                                   CHUNK_ROWS)],
                    buf, sem)
        return carry

    lax.fori_loop(0, NCHUNKS // 2, pair_body, 0)

    pltpu.sync_copy(out_v, out_hbm.at[pl.ds(base_row, ROWS_PER_W)])


def _tc_body(x_ref, m_ref, o_ref):
    xb = x_ref[...]
    hi = xb.astype(jnp.bfloat16)
    lo = (xb - hi.astype(jnp.float32)).astype(jnp.bfloat16)
    mb = m_ref[...]
    o_ref[...] = (
        jnp.dot(hi, mb, preferred_element_type=jnp.float32)
        + jnp.dot(lo, mb, preferred_element_type=jnp.float32))


def _tc_group_sum(x, m):
    return pl.pallas_call(
        _tc_body,
        grid=((ROWS - SC_ROWS) // TBR,),
        in_specs=[
            pl.BlockSpec((TBR, COLS), lambda i: (i + SC_ROWS // TBR, 0)),
            pl.BlockSpec((COLS, GROUPS), lambda i: (0, 0)),
        ],
        out_specs=pl.BlockSpec((TBR, GROUPS), lambda i: (i, 0)),
        out_shape=jax.ShapeDtypeStruct((ROWS - SC_ROWS, GROUPS), jnp.float32),
    )(x, m)


def kernel(x):
    col = jnp.arange(COLS)
    gid = 2 * (col // PAIR_W) + (col % PAIR_W >= 64).astype(jnp.int32)
    m = (gid[:, None] == jnp.arange(GROUPS)[None, :]).astype(jnp.bfloat16)
    y_sc = _sc_group_sum(x)
    y_tc = _tc_group_sum(x, m)
    return jnp.concatenate([y_sc, y_tc], axis=0)
